# Initial kernel scaffold; baseline (speedup 1.0000x reference)
import jax
import jax.numpy as jnp
from jax.experimental import pallas as pl


def kernel(z, x, ln_gamma, ln_beta, w_in, b_in, complex_weight, w_mid, b_mid, gs_xyz, gs_scaling, gs_rotation, gs_features, gs_opacity, w_proj, b_proj):
    raise NotImplementedError("write your pallas kernel here")


# speedup vs baseline: 4.9974x; 4.9974x over previous
import math
import functools
import numpy as np
import jax
import jax.numpy as jnp
from jax import lax
from jax.experimental import pallas as pl
from jax.experimental.pallas import tpu as pltpu
from jax.experimental.pallas import tpu_sc as plsc

_B = 32
_NT = 64
_NS = 1024
_C = 768
_TOPK = 3
_HID2 = 4 * _C
_HID = 2 * _C
_NSMP = _B * _TOPK
_ROWS = _NSMP * 16
_SB = 4
_KB = 6
_BK = _HID2 // _KB
_BR = _ROWS // _SB
_GR = _NSMP * 4 // _SB


def _resize_matrix():
    r1 = np.zeros((4, 2), np.float64)
    for i in range(4):
        src = (i + 0.5) / 2.0 - 0.5
        w0 = max(0.0, 1.0 - abs(src))
        w1 = max(0.0, 1.0 - abs(src - 1.0))
        r1[i] = [w0 / (w0 + w1), w1 / (w0 + w1)]
    m = np.zeros((16, 4), np.float32)
    for i in range(4):
        for j in range(4):
            for a in range(2):
                for b in range(2):
                    m[i * 4 + j, a * 2 + b] = r1[i, a] * r1[j, b]
    return m


_MBIG = np.kron(np.eye(_BR // 16, dtype=np.float32), _resize_matrix())


def _irfft_basis():
    t = np.zeros((16, 24), np.float32)
    for k1 in range(4):
        for k2 in range(3):
            for p in range(2):
                e = np.zeros((4, 3), np.complex128)
                e[k1, k2] = 1.0 if p == 0 else 1j
                h = np.fft.irfft2(e, s=(4, 4))
                t[:, (k1 * 3 + k2) * 2 + p] = h.reshape(16)
    return t


_T2D = _irfft_basis()

_tok = np.arange(_NS)
_AT = np.zeros((_NS, 256), np.float32)
_AT[_tok, (_tok // 64) * 16 + (_tok % 32) // 2] = 0.25


def _scores_body(z_ref, x_ref, at_ref, o_ref):
    z2 = z_ref[0]
    zmax = jnp.max(z2, axis=0, keepdims=True)
    znorm = jnp.sqrt(jnp.sum(zmax * zmax))
    zn = zmax / jnp.maximum(znorm, 1e-12)
    x2 = x_ref[0]
    simt = lax.dot_general(zn, x2, (((1,), (1,)), ((), ())),
                           preferred_element_type=jnp.float32)
    sq = x2 * x2
    ssq = lax.dot_general(jnp.ones((1, _C), jnp.float32), sq,
                          (((1,), (1,)), ((), ())),
                          preferred_element_type=jnp.float32)
    simn = simt / jnp.maximum(jnp.sqrt(ssq), 1e-12)
    o_ref[0] = jnp.dot(simn, at_ref[...],
                       preferred_element_type=jnp.float32)


def _scores(z, x, at):
    return pl.pallas_call(
        _scores_body,
        grid=(_B,),
        in_specs=[
            pl.BlockSpec((1, _NT, _C), lambda b: (b, 0, 0)),
            pl.BlockSpec((1, _NS, _C), lambda b: (b, 0, 0)),
            pl.BlockSpec((_NS, 256), lambda b: (0, 0)),
        ],
        out_specs=pl.BlockSpec((1, 1, 256), lambda b: (b, 0, 0)),
        out_shape=jax.ShapeDtypeStruct((_B, 1, 256), jnp.float32),
    )(z, x, at)


def _bcast_lane(v, lane_idx):
    idx = jnp.full((16, 1), lane_idx, jnp.int32)
    dnums = lax.GatherDimensionNumbers(
        offset_dims=(), collapsed_slice_dims=(0,), start_index_map=(0,))
    return lax.gather(v, idx, dnums, (1,),
                      mode=lax.GatherScatterMode.PROMISE_IN_BOUNDS)


def _topk_gather_body(wsc_hbm, x_hbm, out_hbm, sc_v, rows_v, sem):
    b = lax.axis_index("s") * 2 + lax.axis_index("c")
    pltpu.sync_copy(wsc_hbm.at[b], sc_v)
    lane = lax.broadcasted_iota(jnp.int32, (16,), 0)
    rows = [sc_v[pl.ds(r * 16, 16)] for r in range(16)]
    ids = [lane + r * 16 for r in range(16)]
    wids = []
    for _ in range(_TOPK):
        m = rows[0]
        ra = jnp.zeros((16,), jnp.int32)
        for r in range(1, 16):
            upd = rows[r] > m
            m = jnp.where(upd, rows[r], m)
            ra = jnp.where(upd, r, ra)
        wid_lane = ra * 16 + lane
        s_all = _bcast_lane(plsc.cummax(m), 15)
        cand = jnp.where(m == s_all, wid_lane, 4096)
        wid = -_bcast_lane(plsc.cummax(-cand), 15)
        wids.append(wid)
        for r in range(16):
            rows[r] = jnp.where(ids[r] == wid, -1e30, rows[r])
    t00s = [(w // 16) * 64 + (w % 16) * 2 for w in wids]
    jl = lane % 4
    off = (jl % 2) + 32 * (jl // 2)
    t00v = jnp.where(lane < 4, t00s[0], jnp.where(lane < 8, t00s[1], t00s[2]))
    idx = b * _NS + t00v + off
    pltpu.async_copy(x_hbm.at[idx], rows_v, sem).wait()
    pltpu.sync_copy(rows_v, out_hbm.at[b])


@functools.partial(jax.jit, static_argnums=())
def _topk_gather(wsc, x2d):
    mesh = plsc.VectorSubcoreMesh(core_axis_name="c", subcore_axis_name="s")
    return pl.kernel(
        _topk_gather_body,
        out_type=jax.ShapeDtypeStruct((_B, 16, _C), jnp.float32),
        mesh=mesh,
        scratch_types=[
            pltpu.VMEM((256,), jnp.float32),
            pltpu.VMEM((16, _C), jnp.float32),
            pltpu.SemaphoreType.DMA,
        ],
        compiler_params=pltpu.CompilerParams(needs_layout_passes=False),
    )(wsc, x2d)


def _weights_body(cw_ref, t2d_ref, xyz_ref, scal_ref, rot_ref, feat_ref,
                  opac_ref, wp_ref, bp_ref, h_ref, gs_ref):
    h_ref[...] = jnp.dot(t2d_ref[...], cw_ref[...],
                         preferred_element_type=jnp.float32)
    xy = jnp.tanh(xyz_ref[...])
    mean_x = 0.5 * (xy[0:1] + 1.0) * 4.0
    mean_y = 0.5 * (xy[1:2] + 1.0) * 4.0
    scale = jnp.abs(scal_ref[...] + 0.5)
    s0sq = scale[0:1] * scale[0:1]
    s1sq = scale[1:2] * scale[1:2]
    theta = jax.nn.sigmoid(rot_ref[...]) * (2.0 * math.pi)
    cs = jnp.cos(theta)
    sn = jnp.sin(theta)
    a = cs * cs * s0sq + sn * sn * s1sq
    bb = cs * sn * (s0sq - s1sq)
    c = sn * sn * s0sq + cs * cs * s1sq
    det = jnp.maximum(a * c - bb * bb, 1e-12)
    ca = c / det
    cb = -bb / det
    cc = a / det
    srow = lax.broadcasted_iota(jnp.int32, (16, 9), 0)
    xs = (srow % 4).astype(jnp.float32) + 0.5
    ys = (srow // 4).astype(jnp.float32) + 0.5
    dx = xs - mean_x
    dy = ys - mean_y
    power = -0.5 * (ca * dx * dx + 2.0 * cb * dx * dy + cc * dy * dy)
    alpha = opac_ref[...] * jnp.exp(power)
    img = jnp.clip(jnp.dot(alpha, feat_ref[...],
                           preferred_element_type=jnp.float32), 0.0, 1.0)
    gs_ref[...] = jnp.dot(img, wp_ref[...],
                          preferred_element_type=jnp.float32) + bp_ref[...]


def _weights(cw2d, t2d, xyz_t, scal_t, rot_t, feats, opac_t, wp_t, bp):
    return pl.pallas_call(
        _weights_body,
        in_specs=[pl.BlockSpec(a.shape, lambda: tuple(0 for _ in a.shape))
                  for a in (cw2d, t2d, xyz_t, scal_t, rot_t, feats, opac_t,
                            wp_t, bp)],
        out_specs=[
            pl.BlockSpec((16, _HID2), lambda: (0, 0)),
            pl.BlockSpec((16, _C), lambda: (0, 0)),
        ],
        out_shape=[
            jax.ShapeDtypeStruct((16, _HID2), jnp.float32),
            jax.ShapeDtypeStruct((16, _C), jnp.float32),
        ],
    )(cw2d, t2d, xyz_t, scal_t, rot_t, feats, opac_t, wp_t, bp)


def _dense_body(g_ref, mbig_ref, lng_ref, lnb_ref, w1_ref, b1_ref, h_ref,
                w2_ref, b2_ref, gs_ref, o_ref, acc_ref):
    kb = pl.program_id(1)
    xr = jnp.dot(mbig_ref[...], g_ref[...],
                 preferred_element_type=jnp.float32)
    mu = jnp.mean(xr, axis=1, keepdims=True)
    d = xr - mu
    var = jnp.mean(d * d, axis=1, keepdims=True)
    xn = d * lax.rsqrt(var + 1e-5) * lng_ref[...] + lnb_ref[...]
    y = jnp.dot(xn, w1_ref[...],
                preferred_element_type=jnp.float32) + b1_ref[...]
    y3 = y.reshape(_BR // 16, 16, _BK)
    h = h_ref[...]
    jmask = (lax.broadcasted_iota(jnp.int32, (1, 16, 1), 1) % 4)
    conv = None
    for m in range(16):
        if m == 0:
            cm = y3
        else:
            cm = jnp.concatenate([y3[:, 16 - m:, :], y3[:, :16 - m, :]],
                                 axis=1)
        da1, db1 = m // 4, m % 4
        da0 = (da1 + 1) % 4
        ha = h[da1 * 4 + db1].reshape(1, 1, _BK)
        hb = h[da0 * 4 + db1].reshape(1, 1, _BK)
        gm = jnp.where(jmask >= db1, ha, hb)
        term = cm * gm
        conv = term if conv is None else conv + term
    conv2 = conv.reshape(_BR, _BK)
    part = jnp.dot(conv2, w2_ref[...],
                   preferred_element_type=jnp.float32)

    @pl.when(kb == 0)
    def _():
        acc_ref[...] = part

    @pl.when(kb > 0)
    def _():
        acc_ref[...] += part

    @pl.when(kb == _KB - 1)
    def _():
        t = acc_ref[...] + b2_ref[...]
        x1 = t[:, :_C]
        x2 = t[:, _C:]
        glu = 0.5 * x1 * (1.0 + lax.erf(x1 * (1.0 / math.sqrt(2.0)))) * x2
        o_ref[...] = glu + xr + gs_ref[...]


def _dense(g, mbig, lng, lnb, w1t, b1, h, w2t, b2, gs_t):
    return pl.pallas_call(
        _dense_body,
        grid=(_SB, _KB),
        in_specs=[
            pl.BlockSpec((_GR, _C), lambda sb, kb: (sb, 0)),
            pl.BlockSpec((_BR, _GR), lambda sb, kb: (0, 0)),
            pl.BlockSpec((1, _C), lambda sb, kb: (0, 0)),
            pl.BlockSpec((1, _C), lambda sb, kb: (0, 0)),
            pl.BlockSpec((_C, _BK), lambda sb, kb: (0, kb)),
            pl.BlockSpec((1, _BK), lambda sb, kb: (0, kb)),
            pl.BlockSpec((16, _BK), lambda sb, kb: (0, kb)),
            pl.BlockSpec((_BK, _HID), lambda sb, kb: (kb, 0)),
            pl.BlockSpec((1, _HID), lambda sb, kb: (0, 0)),
            pl.BlockSpec((_BR, _C), lambda sb, kb: (0, 0)),
        ],
        out_specs=pl.BlockSpec((_BR, _C), lambda sb, kb: (sb, 0)),
        out_shape=jax.ShapeDtypeStruct((_ROWS, _C), jnp.float32),
        scratch_shapes=[pltpu.VMEM((_BR, _HID), jnp.float32)],
        compiler_params=pltpu.CompilerParams(
            dimension_semantics=("arbitrary", "arbitrary")),
    )(g, mbig, lng, lnb, w1t, b1, h, w2t, b2, gs_t)


def kernel(z, x, ln_gamma, ln_beta, w_in, b_in, complex_weight, w_mid, b_mid,
           gs_xyz, gs_scaling, gs_rotation, gs_features, gs_opacity, w_proj,
           b_proj):
    at = jnp.asarray(_AT)
    wsc = _scores(z, x, at).reshape(_B, 256)
    rows = _topk_gather(wsc, x.reshape(_B * _NS, _C))[:, :12]
    cw2d = jnp.transpose(complex_weight, (0, 1, 3, 2)).reshape(24, _HID2)
    h, gs16 = _weights(
        cw2d, jnp.asarray(_T2D),
        gs_xyz.T, gs_scaling.T, gs_rotation.T, gs_features,
        gs_opacity.T, w_proj.T, b_proj.reshape(1, _C))
    g = rows.reshape(_NSMP * 4, _C)
    gs_t = jnp.tile(gs16, (_BR // 16, 1))
    out = _dense(g, jnp.asarray(_MBIG), ln_gamma.reshape(1, _C),
                 ln_beta.reshape(1, _C), w_in.T, b_in.reshape(1, _HID2), h,
                 w_mid.T, b_mid.reshape(1, _HID), gs_t)
    return out.reshape(_B, _TOPK * 16, _C)


# speedup vs baseline: 5.8429x; 1.1692x over previous
import math
import functools
import numpy as np
import jax
import jax.numpy as jnp
from jax import lax
from jax.experimental import pallas as pl
from jax.experimental.pallas import tpu as pltpu
from jax.experimental.pallas import tpu_sc as plsc

_B = 32
_NT = 64
_NS = 1024
_C = 768
_TOPK = 3
_HID2 = 4 * _C
_HID = 2 * _C
_NSMP = _B * _TOPK
_ROWS = _NSMP * 16
_SB = 4
_KB = 6
_BK = _HID2 // _KB
_BR = _ROWS // _SB
_GR = _NSMP * 4 // _SB


def _resize_matrix():
    r1 = np.zeros((4, 2), np.float64)
    for i in range(4):
        src = (i + 0.5) / 2.0 - 0.5
        w0 = max(0.0, 1.0 - abs(src))
        w1 = max(0.0, 1.0 - abs(src - 1.0))
        r1[i] = [w0 / (w0 + w1), w1 / (w0 + w1)]
    m = np.zeros((16, 4), np.float32)
    for i in range(4):
        for j in range(4):
            for a in range(2):
                for b in range(2):
                    m[i * 4 + j, a * 2 + b] = r1[i, a] * r1[j, b]
    return m


_MBIG = np.kron(np.eye(_BR // 16, dtype=np.float32), _resize_matrix())


def _irfft_basis():
    t = np.zeros((16, 24), np.float32)
    for k1 in range(4):
        for k2 in range(3):
            for p in range(2):
                e = np.zeros((4, 3), np.complex128)
                e[k1, k2] = 1.0 if p == 0 else 1j
                h = np.fft.irfft2(e, s=(4, 4))
                t[:, (k1 * 3 + k2) * 2 + p] = h.reshape(16)
    return t


_T2D = _irfft_basis()

_tok = np.arange(_NS)
_AT = np.zeros((_NS, 256), np.float32)
_AT[_tok, (_tok // 64) * 16 + (_tok % 32) // 2] = 0.25


def _scores_body(z_ref, x_ref, at_ref, o_ref):
    z2 = z_ref[0]
    zmax = jnp.max(z2, axis=0, keepdims=True)
    znorm = jnp.sqrt(jnp.sum(zmax * zmax))
    zn = zmax / jnp.maximum(znorm, 1e-12)
    x2 = x_ref[0]
    simt = lax.dot_general(zn, x2, (((1,), (1,)), ((), ())),
                           preferred_element_type=jnp.float32)
    sq = x2 * x2
    ssq = lax.dot_general(jnp.ones((1, _C), jnp.float32), sq,
                          (((1,), (1,)), ((), ())),
                          preferred_element_type=jnp.float32)
    simn = simt / jnp.maximum(jnp.sqrt(ssq), 1e-12)
    o_ref[0] = jnp.dot(simn, at_ref[...],
                       preferred_element_type=jnp.float32)


def _scores(z, x, at):
    return pl.pallas_call(
        _scores_body,
        grid=(_B,),
        in_specs=[
            pl.BlockSpec((1, _NT, _C), lambda b: (b, 0, 0)),
            pl.BlockSpec((1, _NS, _C), lambda b: (b, 0, 0)),
            pl.BlockSpec((_NS, 256), lambda b: (0, 0)),
        ],
        out_specs=pl.BlockSpec((1, 1, 256), lambda b: (b, 0, 0)),
        out_shape=jax.ShapeDtypeStruct((_B, 1, 256), jnp.float32),
    )(z, x, at)


def _bcast_lane(v, lane_idx):
    idx = jnp.full((16, 1), lane_idx, jnp.int32)
    dnums = lax.GatherDimensionNumbers(
        offset_dims=(), collapsed_slice_dims=(0,), start_index_map=(0,))
    return lax.gather(v, idx, dnums, (1,),
                      mode=lax.GatherScatterMode.PROMISE_IN_BOUNDS)


def _topk_gather_body(wsc_hbm, x_hbm, out_hbm, sc_v, rows_v, sem):
    b = lax.axis_index("s") * 2 + lax.axis_index("c")
    pltpu.sync_copy(wsc_hbm.at[b], sc_v)
    lane = lax.broadcasted_iota(jnp.int32, (16,), 0)
    rows = [sc_v[pl.ds(r * 16, 16)] for r in range(16)]
    ids = [lane + r * 16 for r in range(16)]
    wids = []
    for _ in range(_TOPK):
        m = rows[0]
        ra = jnp.zeros((16,), jnp.int32)
        for r in range(1, 16):
            upd = rows[r] > m
            m = jnp.where(upd, rows[r], m)
            ra = jnp.where(upd, r, ra)
        wid_lane = ra * 16 + lane
        s_all = _bcast_lane(plsc.cummax(m), 15)
        cand = jnp.where(m == s_all, wid_lane, 4096)
        wid = -_bcast_lane(plsc.cummax(-cand), 15)
        wids.append(wid)
        for r in range(16):
            rows[r] = jnp.where(ids[r] == wid, -1e30, rows[r])
    t00s = [(w // 16) * 64 + (w % 16) * 2 for w in wids]
    jl = lane % 4
    off = (jl % 2) + 32 * (jl // 2)
    t00v = jnp.where(lane < 4, t00s[0], jnp.where(lane < 8, t00s[1], t00s[2]))
    idx = b * _NS + t00v + off
    pltpu.async_copy(x_hbm.at[idx], rows_v, sem).wait()
    pltpu.sync_copy(rows_v, out_hbm.at[b])


@functools.partial(jax.jit, static_argnums=())
def _topk_gather(wsc, x2d):
    mesh = plsc.VectorSubcoreMesh(core_axis_name="c", subcore_axis_name="s")
    return pl.kernel(
        _topk_gather_body,
        out_type=jax.ShapeDtypeStruct((_B, 16, _C), jnp.float32),
        mesh=mesh,
        scratch_types=[
            pltpu.VMEM((256,), jnp.float32),
            pltpu.VMEM((16, _C), jnp.float32),
            pltpu.SemaphoreType.DMA,
        ],
        compiler_params=pltpu.CompilerParams(needs_layout_passes=False),
    )(wsc, x2d)


def _weights_body(cw_ref, t2d_ref, xyz_ref, scal_ref, rot_ref, feat_ref,
                  opac_ref, wp_ref, bp_ref, h_ref, gs_ref):
    h_ref[...] = jnp.dot(t2d_ref[...], cw_ref[...],
                         preferred_element_type=jnp.float32)
    xy = jnp.tanh(xyz_ref[...])
    mean_x = 0.5 * (xy[0:1] + 1.0) * 4.0
    mean_y = 0.5 * (xy[1:2] + 1.0) * 4.0
    scale = jnp.abs(scal_ref[...] + 0.5)
    s0sq = scale[0:1] * scale[0:1]
    s1sq = scale[1:2] * scale[1:2]
    theta = jax.nn.sigmoid(rot_ref[...]) * (2.0 * math.pi)
    cs = jnp.cos(theta)
    sn = jnp.sin(theta)
    a = cs * cs * s0sq + sn * sn * s1sq
    bb = cs * sn * (s0sq - s1sq)
    c = sn * sn * s0sq + cs * cs * s1sq
    det = jnp.maximum(a * c - bb * bb, 1e-12)
    ca = c / det
    cb = -bb / det
    cc = a / det
    srow = lax.broadcasted_iota(jnp.int32, (16, 9), 0)
    xs = (srow % 4).astype(jnp.float32) + 0.5
    ys = (srow // 4).astype(jnp.float32) + 0.5
    dx = xs - mean_x
    dy = ys - mean_y
    power = -0.5 * (ca * dx * dx + 2.0 * cb * dx * dy + cc * dy * dy)
    alpha = opac_ref[...] * jnp.exp(power)
    img = jnp.clip(jnp.dot(alpha, feat_ref[...],
                           preferred_element_type=jnp.float32), 0.0, 1.0)
    gs_ref[...] = jnp.dot(img, wp_ref[...],
                          preferred_element_type=jnp.float32) + bp_ref[...]


def _weights(cw2d, t2d, xyz_t, scal_t, rot_t, feats, opac_t, wp_t, bp):
    return pl.pallas_call(
        _weights_body,
        in_specs=[pl.BlockSpec(a.shape, lambda: tuple(0 for _ in a.shape))
                  for a in (cw2d, t2d, xyz_t, scal_t, rot_t, feats, opac_t,
                            wp_t, bp)],
        out_specs=[
            pl.BlockSpec((16, _HID2), lambda: (0, 0)),
            pl.BlockSpec((16, _C), lambda: (0, 0)),
        ],
        out_shape=[
            jax.ShapeDtypeStruct((16, _HID2), jnp.float32),
            jax.ShapeDtypeStruct((16, _C), jnp.float32),
        ],
    )(cw2d, t2d, xyz_t, scal_t, rot_t, feats, opac_t, wp_t, bp)


def _dense_body(g_ref, mbig_ref, lng_ref, lnb_ref, w1_ref, b1_ref, h_ref,
                w2_ref, b2_ref, gs_ref, o_ref, acc_ref):
    kb = pl.program_id(1)
    xr = jnp.dot(mbig_ref[...], g_ref[...],
                 preferred_element_type=jnp.float32)
    mu = jnp.mean(xr, axis=1, keepdims=True)
    d = xr - mu
    var = jnp.mean(d * d, axis=1, keepdims=True)
    xn = d * lax.rsqrt(var + 1e-5) * lng_ref[...] + lnb_ref[...]
    y = lax.dot_general(xn, w1_ref[...], (((1,), (1,)), ((), ())),
                        preferred_element_type=jnp.float32) + b1_ref[...]
    y3 = y.reshape(_BR // 16, 16, _BK)
    h = h_ref[...]
    jmask = (lax.broadcasted_iota(jnp.int32, (1, 16, 1), 1) % 4)
    conv = None
    for m in range(16):
        if m == 0:
            cm = y3
        else:
            cm = jnp.concatenate([y3[:, 16 - m:, :], y3[:, :16 - m, :]],
                                 axis=1)
        da1, db1 = m // 4, m % 4
        da0 = (da1 + 1) % 4
        ha = h[da1 * 4 + db1].reshape(1, 1, _BK)
        hb = h[da0 * 4 + db1].reshape(1, 1, _BK)
        gm = jnp.where(jmask >= db1, ha, hb)
        term = cm * gm
        conv = term if conv is None else conv + term
    conv2 = conv.reshape(_BR, _BK)
    part = lax.dot_general(conv2, w2_ref[...], (((1,), (1,)), ((), ())),
                           preferred_element_type=jnp.float32)

    @pl.when(kb == 0)
    def _():
        acc_ref[...] = part

    @pl.when(kb > 0)
    def _():
        acc_ref[...] += part

    @pl.when(kb == _KB - 1)
    def _():
        t = acc_ref[...] + b2_ref[...]
        x1 = t[:, :_C]
        x2 = t[:, _C:]
        glu = 0.5 * x1 * (1.0 + lax.erf(x1 * (1.0 / math.sqrt(2.0)))) * x2
        o_ref[...] = glu + xr + gs_ref[...]


def _dense(g, mbig, lng, lnb, w1t, b1, h, w2t, b2, gs_t):
    return pl.pallas_call(
        _dense_body,
        grid=(_SB, _KB),
        in_specs=[
            pl.BlockSpec((_GR, _C), lambda sb, kb: (sb, 0)),
            pl.BlockSpec((_BR, _GR), lambda sb, kb: (0, 0)),
            pl.BlockSpec((1, _C), lambda sb, kb: (0, 0)),
            pl.BlockSpec((1, _C), lambda sb, kb: (0, 0)),
            pl.BlockSpec((_BK, _C), lambda sb, kb: (kb, 0)),
            pl.BlockSpec((1, _BK), lambda sb, kb: (0, kb)),
            pl.BlockSpec((16, _BK), lambda sb, kb: (0, kb)),
            pl.BlockSpec((_HID, _BK), lambda sb, kb: (0, kb)),
            pl.BlockSpec((1, _HID), lambda sb, kb: (0, 0)),
            pl.BlockSpec((_BR, _C), lambda sb, kb: (0, 0)),
        ],
        out_specs=pl.BlockSpec((_BR, _C), lambda sb, kb: (sb, 0)),
        out_shape=jax.ShapeDtypeStruct((_ROWS, _C), jnp.float32),
        scratch_shapes=[pltpu.VMEM((_BR, _HID), jnp.float32)],
        compiler_params=pltpu.CompilerParams(
            dimension_semantics=("arbitrary", "arbitrary")),
    )(g, mbig, lng, lnb, w1t, b1, h, w2t, b2, gs_t)


def kernel(z, x, ln_gamma, ln_beta, w_in, b_in, complex_weight, w_mid, b_mid,
           gs_xyz, gs_scaling, gs_rotation, gs_features, gs_opacity, w_proj,
           b_proj):
    at = jnp.asarray(_AT)
    wsc = _scores(z, x, at).reshape(_B, 256)
    rows = _topk_gather(wsc, x.reshape(_B * _NS, _C))[:, :12]
    cw2d = jnp.transpose(complex_weight, (0, 1, 3, 2)).reshape(24, _HID2)
    h, gs16 = _weights(
        cw2d, jnp.asarray(_T2D),
        gs_xyz.T, gs_scaling.T, gs_rotation.T, gs_features,
        gs_opacity.T, w_proj.T, b_proj.reshape(1, _C))
    g = rows.reshape(_NSMP * 4, _C)
    gs_t = jnp.tile(gs16, (_BR // 16, 1))
    out = _dense(g, jnp.asarray(_MBIG), ln_gamma.reshape(1, _C),
                 ln_beta.reshape(1, _C), w_in, b_in.reshape(1, _HID2), h,
                 w_mid, b_mid.reshape(1, _HID), gs_t)
    return out.reshape(_B, _TOPK * 16, _C)
